# Initial kernel scaffold; baseline (speedup 1.0000x reference)
import jax
import jax.numpy as jnp
from jax.experimental import pallas as pl


def kernel(x, edge_index, edge_attr, index, W1, b1, W2, b2):
    raise NotImplementedError("write your pallas kernel here")


# speedup vs baseline: 10.5555x; 10.5555x over previous
import functools

import jax
import jax.numpy as jnp
from jax import lax
from jax.experimental import pallas as pl
from jax.experimental.pallas import tpu as pltpu
from jax.experimental.pallas import tpu_sc as plsc

N = 10000
E = 320000
D = 128
H2 = 64
NQ = 1000

NP = 10240
BQ = 1024

L = 16
NC = 2
NS = 16
NW = NC * NS
EW = E // NW
CE = 80
NCH = EW // CE
RPW = NP // NS
QW = BQ // NW

BN = 2048


def _mesh():
    return plsc.VectorSubcoreMesh(core_axis_name="c", subcore_axis_name="s")


def _worker_ids():
    cid = lax.axis_index("c")
    sid = lax.axis_index("s")
    return cid, sid, cid * NS + sid


def _deg_body(col3, w3, deg2, deg_sh, colv, wv, zbuf, sem):
    cid, sid, wid = _worker_ids()
    pltpu.sync_copy(col3.at[wid], colv)
    pltpu.sync_copy(w3.at[wid], wv)

    def zb(j, c):
        zbuf[pl.ds(j * L, L)] = jnp.zeros((L,), jnp.float32)
        return c

    lax.fori_loop(0, RPW // L, zb, 0)
    pltpu.sync_copy(zbuf, deg_sh.at[pl.ds(sid * RPW, RPW)])
    plsc.subcore_barrier()

    def chunk(k, c):
        pltpu.sync_copy(wv.at[k], deg_sh.at[colv.at[k]], add=True)
        return c

    lax.fori_loop(0, NCH, chunk, 0)
    plsc.subcore_barrier()
    pltpu.sync_copy(deg_sh.at[pl.ds(sid * RPW, RPW)],
                    deg2.at[cid, pl.ds(sid * RPW, RPW)])


def _deg_call(col3, w3):
    f = pl.kernel(
        _deg_body,
        out_type=jax.ShapeDtypeStruct((NC, NP), jnp.float32),
        mesh=_mesh(),
        scratch_types=[
            pltpu.VMEM_SHARED((NP,), jnp.float32),
            pltpu.VMEM((NCH, CE), jnp.int32),
            pltpu.VMEM((NCH, CE), jnp.float32),
            pltpu.VMEM((RPW,), jnp.float32),
            pltpu.SemaphoreType.DMA,
        ],
    )
    return f(col3, w3)


def _make_agg_body(H):
    RJ = H // L

    def body(g_hbm, row3, col3, w3, agg2, acc_sh, rowv, colv, wv, rows, sem):
        cid, sid, wid = _worker_ids()

        def zrow(r, c):
            for j in range(RJ):
                rows[r, pl.ds(j * L, L)] = jnp.zeros((L,), jnp.float32)
            return c

        lax.fori_loop(0, CE, zrow, 0)

        def zcp(t, c):
            pltpu.sync_copy(rows, acc_sh.at[pl.ds(sid * RPW + t * CE, CE)])
            return c

        lax.fori_loop(0, RPW // CE, zcp, 0)
        plsc.subcore_barrier()

        def chunk(k, c):
            pltpu.sync_copy(row3.at[wid, k], rowv)
            pltpu.sync_copy(col3.at[wid, k], colv)
            pltpu.sync_copy(w3.at[wid, k], wv)
            pltpu.async_copy(g_hbm.at[rowv], rows, sem).wait()

            def scale(g, cc):
                wvec = wv[pl.ds(g * L, L)]
                for t in range(L):
                    ws = wvec[t]
                    e = g * L + t
                    for j in range(RJ):
                        sl = pl.ds(j * L, L)
                        rows[e, sl] = rows[e, sl] * ws
                return cc

            lax.fori_loop(0, CE // L, scale, 0)
            pltpu.sync_copy(rows, acc_sh.at[colv], add=True)
            return c

        lax.fori_loop(0, NCH, chunk, 0)
        plsc.subcore_barrier()
        pltpu.sync_copy(acc_sh.at[pl.ds(sid * RPW, RPW)],
                        agg2.at[cid, pl.ds(sid * RPW, RPW)])

    return body


def _agg_call(g, row3, col3, w3, H):
    f = pl.kernel(
        _make_agg_body(H),
        out_type=jax.ShapeDtypeStruct((NC, NP, H), jnp.float32),
        mesh=_mesh(),
        scratch_types=[
            pltpu.VMEM_SHARED((NP, H), jnp.float32),
            pltpu.VMEM((CE,), jnp.int32),
            pltpu.VMEM((CE,), jnp.int32),
            pltpu.VMEM((CE,), jnp.float32),
            pltpu.VMEM((CE, H), jnp.float32),
            pltpu.SemaphoreType.DMA,
        ],
    )
    return f(g, row3, col3, w3)


def _fin_body(a2, b2c, g2, dinv_hbm, b2_hbm, idxp, out,
              idxv, ra, rb, rg, dinvv, bv, ob, sem):
    cid, sid, wid = _worker_ids()
    base = wid * QW
    pltpu.sync_copy(idxp.at[pl.ds(base, QW)], idxv)
    pltpu.sync_copy(b2_hbm, bv)
    pltpu.sync_copy(dinv_hbm, dinvv)
    c1 = pltpu.async_copy(a2.at[idxv], ra, sem)
    c2 = pltpu.async_copy(b2c.at[idxv], rb, sem)
    c3 = pltpu.async_copy(g2.at[idxv], rg, sem)
    c1.wait()
    c2.wait()
    c3.wait()

    def rowg(g, c):
        ivec = idxv[pl.ds(g * L, L)]
        dvals = plsc.load_gather(dinvv, [ivec])
        for t in range(L):
            dv = dvals[t]
            r = g * L + t
            for j in range(D // L):
                sl = pl.ds(j * L, L)
                v = dv * (ra[r, sl] + rb[r, sl] + rg[r, sl]) + bv[sl]
                ob[r, sl] = jnp.maximum(v, 0.0)
        return c

    lax.fori_loop(0, QW // L, rowg, 0)
    pltpu.sync_copy(ob, out.at[pl.ds(base, QW)])


def _fin_call(a2, b2c, g2, dinv, b2, idxp):
    f = pl.kernel(
        _fin_body,
        out_type=jax.ShapeDtypeStruct((BQ, D), jnp.float32),
        mesh=_mesh(),
        scratch_types=[
            pltpu.VMEM((QW,), jnp.int32),
            pltpu.VMEM((QW, D), jnp.float32),
            pltpu.VMEM((QW, D), jnp.float32),
            pltpu.VMEM((QW, D), jnp.float32),
            pltpu.VMEM((NP,), jnp.float32),
            pltpu.VMEM((D,), jnp.float32),
            pltpu.VMEM((QW, D), jnp.float32),
            pltpu.SemaphoreType.DMA,
        ],
        compiler_params=pltpu.CompilerParams(needs_layout_passes=False),
    )
    return f(a2, b2c, g2, dinv, b2, idxp)


def _mm1_body(x_ref, w1_ref, dga_ref, dgb_ref, dinv_ref, g1_ref):
    deg = dga_ref[...] + dgb_ref[...] + 1.0
    dinv = jnp.where(deg > 0, lax.rsqrt(jnp.maximum(deg, 1e-12)), 0.0)
    dinv_ref[...] = dinv
    g1_ref[...] = dinv[:, None] * jnp.dot(
        x_ref[...], w1_ref[...], preferred_element_type=jnp.float32)


def _mm1_call(xp, W1, dega, degb):
    return pl.pallas_call(
        _mm1_body,
        grid=(NP // BN,),
        in_specs=[
            pl.BlockSpec((BN, D), lambda i: (i, 0)),
            pl.BlockSpec((D, D), lambda i: (0, 0)),
            pl.BlockSpec((BN,), lambda i: (i,)),
            pl.BlockSpec((BN,), lambda i: (i,)),
        ],
        out_specs=[
            pl.BlockSpec((BN,), lambda i: (i,)),
            pl.BlockSpec((BN, D), lambda i: (i, 0)),
        ],
        out_shape=[
            jax.ShapeDtypeStruct((NP,), jnp.float32),
            jax.ShapeDtypeStruct((NP, D), jnp.float32),
        ],
    )(xp, W1, dega, degb)


def _mm2_body(a_ref, b_ref, g1_ref, dinv_ref, b1_ref, w2_ref, g2_ref):
    dinv = dinv_ref[...][:, None]
    h1 = jnp.maximum(
        dinv * (a_ref[...] + b_ref[...] + g1_ref[...]) + b1_ref[...], 0.0)
    g2_ref[...] = dinv * jnp.dot(
        h1, w2_ref[...], preferred_element_type=jnp.float32)


def _mm2_call(a, b, g1, dinv, b1r, W2):
    return pl.pallas_call(
        _mm2_body,
        grid=(NP // BN,),
        in_specs=[
            pl.BlockSpec((BN, D), lambda i: (i, 0)),
            pl.BlockSpec((BN, D), lambda i: (i, 0)),
            pl.BlockSpec((BN, D), lambda i: (i, 0)),
            pl.BlockSpec((BN,), lambda i: (i,)),
            pl.BlockSpec((1, D), lambda i: (0, 0)),
            pl.BlockSpec((D, D), lambda i: (0, 0)),
        ],
        out_specs=pl.BlockSpec((BN, D), lambda i: (i, 0)),
        out_shape=jax.ShapeDtypeStruct((NP, D), jnp.float32),
    )(a, b, g1, dinv, b1r, W2)


def kernel(x, edge_index, edge_attr, index, W1, b1, W2, b2):
    row3 = edge_index[0].reshape(NW, NCH, CE)
    col3 = edge_index[1].reshape(NW, NCH, CE)
    w3 = edge_attr.reshape(NW, NCH, CE)
    xp = jnp.pad(x, ((0, NP - N), (0, 0)))
    idxp = jnp.pad(index, (0, BQ - NQ))
    b1r = b1.reshape(1, D)
    W2p = jnp.pad(W2, ((0, 0), (0, D - H2)))
    b2p = jnp.pad(b2, (0, D - H2))

    deg2 = _deg_call(col3, w3)
    dinv, g1 = _mm1_call(xp, W1, deg2[0], deg2[1])
    agg1 = _agg_call(g1, row3, col3, w3, D)
    g2 = _mm2_call(agg1[0], agg1[1], g1, dinv, b1r, W2p)
    agg2 = _agg_call(g2, row3, col3, w3, D)
    res = _fin_call(agg2[0], agg2[1], g2, dinv, b2p, idxp)
    return res[:NQ, :H2]


# speedup vs baseline: 25.2151x; 2.3888x over previous
import functools

import jax
import jax.numpy as jnp
from jax import lax
from jax.experimental import pallas as pl
from jax.experimental.pallas import tpu as pltpu
from jax.experimental.pallas import tpu_sc as plsc

N = 10000
E = 320000
D = 128
H2 = 64
NQ = 1000

NP = 10240
BQ = 1024

L = 16
NC = 2
NS = 16
NW = NC * NS
EW = E // NW
CE = 80
NCH = EW // CE
RPW = NP // NS
QW = BQ // NW

BN = 2048


def _mesh():
    return plsc.VectorSubcoreMesh(core_axis_name="c", subcore_axis_name="s")


def _worker_ids():
    cid = lax.axis_index("c")
    sid = lax.axis_index("s")
    return cid, sid, cid * NS + sid


def _deg_body(col3, w3, deg2, deg_sh, colv, wv, zbuf, sem):
    cid, sid, wid = _worker_ids()
    pltpu.sync_copy(col3.at[wid], colv)
    pltpu.sync_copy(w3.at[wid], wv)

    def zb(j, c):
        zbuf[pl.ds(j * L, L)] = jnp.zeros((L,), jnp.float32)
        return c

    lax.fori_loop(0, RPW // L, zb, 0)
    pltpu.sync_copy(zbuf, deg_sh.at[pl.ds(sid * RPW, RPW)])
    plsc.subcore_barrier()

    def chunk(k, c):
        pltpu.sync_copy(wv.at[k], deg_sh.at[colv.at[k]], add=True)
        return c

    lax.fori_loop(0, NCH, chunk, 0)
    plsc.subcore_barrier()
    pltpu.sync_copy(deg_sh.at[pl.ds(sid * RPW, RPW)],
                    deg2.at[cid, pl.ds(sid * RPW, RPW)])


def _deg_call(col3, w3):
    f = pl.kernel(
        _deg_body,
        out_type=jax.ShapeDtypeStruct((NC, NP), jnp.float32),
        mesh=_mesh(),
        scratch_types=[
            pltpu.VMEM_SHARED((NP,), jnp.float32),
            pltpu.VMEM((NCH, CE), jnp.int32),
            pltpu.VMEM((NCH, CE), jnp.float32),
            pltpu.VMEM((RPW,), jnp.float32),
            pltpu.SemaphoreType.DMA,
        ],
    )
    return f(col3, w3)


NB = 4


def _make_agg_body(H):
    RJ = H // L

    def body(g_hbm, ep, agg2, acc_sh,
             eb0, eb1, eb2, eb3, r0, r1, r2, r3,
             gs0, gs1, gs2, gs3, ss0, ss1, ss2, ss3):
        ebs = (eb0, eb1, eb2, eb3)
        rs = (r0, r1, r2, r3)
        gsem = (gs0, gs1, gs2, gs3)
        ssem = (ss0, ss1, ss2, ss3)
        cid, sid, wid = _worker_ids()

        def zrow(r, c):
            for j in range(RJ):
                r0[r, pl.ds(j * L, L)] = jnp.zeros((L,), jnp.float32)
            return c

        lax.fori_loop(0, CE, zrow, 0)

        def zcp(t, c):
            pltpu.sync_copy(r0, acc_sh.at[pl.ds(sid * RPW + t * CE, CE)])
            return c

        lax.fori_loop(0, RPW // CE, zcp, 0)
        plsc.subcore_barrier()

        def stage(k, b):
            pltpu.sync_copy(ep.at[wid, k], ebs[b])
            pltpu.async_copy(g_hbm.at[ebs[b].at[0]], rs[b], gsem[b])

        def wait_gather(b):
            pltpu.make_async_copy(g_hbm.at[ebs[b].at[0]], rs[b],
                                  gsem[b]).wait()

        def scale(b):
            eb, rb = ebs[b], rs[b]

            def grp(g, c):
                wvec = plsc.bitcast(eb[2, pl.ds(g * L, L)], jnp.float32)
                for t in range(L):
                    ws = wvec[t]
                    e = g * L + t
                    for j in range(RJ):
                        sl = pl.ds(j * L, L)
                        rb[e, sl] = rb[e, sl] * ws
                return c

            lax.fori_loop(0, CE // L, grp, 0)

        def start_scatter(b):
            pltpu.async_copy(rs[b], acc_sh.at[ebs[b].at[1]], ssem[b],
                             add=True)

        def wait_scatter(b):
            pltpu.make_async_copy(rs[b], acc_sh.at[ebs[b].at[1]],
                                  ssem[b]).wait()

        def process(b):
            wait_gather(b)
            scale(b)
            start_scatter(b)

        stage(0, 0)
        stage(1, 1)
        stage(2, 2)
        process(0)
        stage(3, 3)
        process(1)
        wait_scatter(0)
        stage(4, 0)
        process(2)
        wait_scatter(1)
        stage(5, 1)
        process(3)

        def round_(r, c):
            for b in range(NB):
                b2 = (b + 2) % NB
                wait_scatter(b2)
                stage(4 * r + b + 2, b2)
                process(b)
            return c

        lax.fori_loop(1, NCH // NB - 1, round_, 0)

        for b in range(NB):
            b2 = (b + 2) % NB
            wait_scatter(b2)
            if 120 + b + 2 < NCH:
                stage(120 + b + 2, b2)
            process(b)
        wait_scatter(2)
        process(0)
        wait_scatter(3)
        wait_scatter(0)

        plsc.subcore_barrier()
        pltpu.sync_copy(acc_sh.at[pl.ds(sid * RPW, RPW)],
                        agg2.at[cid, pl.ds(sid * RPW, RPW)])

    return body


def _agg_call(g, ep, H):
    f = pl.kernel(
        _make_agg_body(H),
        out_type=jax.ShapeDtypeStruct((NC, NP, H), jnp.float32),
        mesh=_mesh(),
        scratch_types=(
            [pltpu.VMEM_SHARED((NP, H), jnp.float32)]
            + [pltpu.VMEM((3, CE), jnp.int32) for _ in range(NB)]
            + [pltpu.VMEM((CE, H), jnp.float32) for _ in range(NB)]
            + [pltpu.SemaphoreType.DMA] * (2 * NB)
        ),
        compiler_params=pltpu.CompilerParams(needs_layout_passes=False),
    )
    return f(g, ep)


def _fin_body(a2, b2c, g2, dinv_hbm, b2_hbm, idxp, out,
              idxv, ra, rb, rg, dinvv, bv, ob, sem):
    cid, sid, wid = _worker_ids()
    base = wid * QW
    pltpu.sync_copy(idxp.at[pl.ds(base, QW)], idxv)
    pltpu.sync_copy(b2_hbm, bv)
    pltpu.sync_copy(dinv_hbm, dinvv)
    c1 = pltpu.async_copy(a2.at[idxv], ra, sem)
    c2 = pltpu.async_copy(b2c.at[idxv], rb, sem)
    c3 = pltpu.async_copy(g2.at[idxv], rg, sem)
    c1.wait()
    c2.wait()
    c3.wait()

    def rowg(g, c):
        ivec = idxv[pl.ds(g * L, L)]
        dvals = plsc.load_gather(dinvv, [ivec])
        for t in range(L):
            dv = dvals[t]
            r = g * L + t
            for j in range(D // L):
                sl = pl.ds(j * L, L)
                v = dv * (ra[r, sl] + rb[r, sl] + rg[r, sl]) + bv[sl]
                ob[r, sl] = jnp.maximum(v, 0.0)
        return c

    lax.fori_loop(0, QW // L, rowg, 0)
    pltpu.sync_copy(ob, out.at[pl.ds(base, QW)])


def _fin_call(a2, b2c, g2, dinv, b2, idxp):
    f = pl.kernel(
        _fin_body,
        out_type=jax.ShapeDtypeStruct((BQ, D), jnp.float32),
        mesh=_mesh(),
        scratch_types=[
            pltpu.VMEM((QW,), jnp.int32),
            pltpu.VMEM((QW, D), jnp.float32),
            pltpu.VMEM((QW, D), jnp.float32),
            pltpu.VMEM((QW, D), jnp.float32),
            pltpu.VMEM((NP,), jnp.float32),
            pltpu.VMEM((D,), jnp.float32),
            pltpu.VMEM((QW, D), jnp.float32),
            pltpu.SemaphoreType.DMA,
        ],
        compiler_params=pltpu.CompilerParams(needs_layout_passes=False),
    )
    return f(a2, b2c, g2, dinv, b2, idxp)


def _mm1_body(x_ref, w1_ref, dga_ref, dgb_ref, dinv_ref, g1_ref):
    deg = dga_ref[...] + dgb_ref[...] + 1.0
    dinv = jnp.where(deg > 0, lax.rsqrt(jnp.maximum(deg, 1e-12)), 0.0)
    dinv_ref[...] = dinv
    g1_ref[...] = dinv[:, None] * jnp.dot(
        x_ref[...], w1_ref[...], preferred_element_type=jnp.float32)


def _mm1_call(xp, W1, dega, degb):
    return pl.pallas_call(
        _mm1_body,
        grid=(NP // BN,),
        in_specs=[
            pl.BlockSpec((BN, D), lambda i: (i, 0)),
            pl.BlockSpec((D, D), lambda i: (0, 0)),
            pl.BlockSpec((BN,), lambda i: (i,)),
            pl.BlockSpec((BN,), lambda i: (i,)),
        ],
        out_specs=[
            pl.BlockSpec((BN,), lambda i: (i,)),
            pl.BlockSpec((BN, D), lambda i: (i, 0)),
        ],
        out_shape=[
            jax.ShapeDtypeStruct((NP,), jnp.float32),
            jax.ShapeDtypeStruct((NP, D), jnp.float32),
        ],
    )(xp, W1, dega, degb)


def _mm2_body(a_ref, b_ref, g1_ref, dinv_ref, b1_ref, w2_ref, g2_ref):
    dinv = dinv_ref[...][:, None]
    h1 = jnp.maximum(
        dinv * (a_ref[...] + b_ref[...] + g1_ref[...]) + b1_ref[...], 0.0)
    g2_ref[...] = dinv * jnp.dot(
        h1, w2_ref[...], preferred_element_type=jnp.float32)


def _mm2_call(a, b, g1, dinv, b1r, W2):
    return pl.pallas_call(
        _mm2_body,
        grid=(NP // BN,),
        in_specs=[
            pl.BlockSpec((BN, D), lambda i: (i, 0)),
            pl.BlockSpec((BN, D), lambda i: (i, 0)),
            pl.BlockSpec((BN, D), lambda i: (i, 0)),
            pl.BlockSpec((BN,), lambda i: (i,)),
            pl.BlockSpec((1, D), lambda i: (0, 0)),
            pl.BlockSpec((D, D), lambda i: (0, 0)),
        ],
        out_specs=pl.BlockSpec((BN, D), lambda i: (i, 0)),
        out_shape=jax.ShapeDtypeStruct((NP, D), jnp.float32),
    )(a, b, g1, dinv, b1r, W2)


def kernel(x, edge_index, edge_attr, index, W1, b1, W2, b2):
    col3 = edge_index[1].reshape(NW, NCH, CE)
    w3 = edge_attr.reshape(NW, NCH, CE)
    w_bits = lax.bitcast_convert_type(edge_attr, jnp.int32)
    ep = jnp.stack([edge_index[0], edge_index[1], w_bits], axis=0)
    ep = ep.reshape(3, NW, NCH, CE).transpose(1, 2, 0, 3)
    xp = jnp.pad(x, ((0, NP - N), (0, 0)))
    idxp = jnp.pad(index, (0, BQ - NQ))
    b1r = b1.reshape(1, D)
    W2p = jnp.pad(W2, ((0, 0), (0, D - H2)))
    b2p = jnp.pad(b2, (0, D - H2))

    deg2 = _deg_call(col3, w3)
    dinv, g1 = _mm1_call(xp, W1, deg2[0], deg2[1])
    agg1 = _agg_call(g1, ep, D)
    g2 = _mm2_call(agg1[0], agg1[1], g1, dinv, b1r, W2p)
    agg2 = _agg_call(g2, ep, D)
    res = _fin_call(agg2[0], agg2[1], g2, dinv, b2p, idxp)
    return res[:NQ, :H2]


# speedup vs baseline: 26.0410x; 1.0328x over previous
import functools

import jax
import jax.numpy as jnp
from jax import lax
from jax.experimental import pallas as pl
from jax.experimental.pallas import tpu as pltpu
from jax.experimental.pallas import tpu_sc as plsc

N = 10000
E = 320000
D = 128
H2 = 64
NQ = 1000

NP = 10240
BQ = 1024

L = 16
NC = 2
NS = 16
NW = NC * NS
EW = E // NW
CE = 80
NCH = EW // CE
RPW = NP // NS
QW = BQ // NW

BN = 2048


def _mesh():
    return plsc.VectorSubcoreMesh(core_axis_name="c", subcore_axis_name="s")


def _worker_ids():
    cid = lax.axis_index("c")
    sid = lax.axis_index("s")
    return cid, sid, cid * NS + sid


def _deg_body(col3, w3, dega, degb, deg_sh, colv, wv, zbuf, sem):
    cid, sid, wid = _worker_ids()
    pltpu.sync_copy(col3.at[wid], colv)
    pltpu.sync_copy(w3.at[wid], wv)

    def zb(j, c):
        zbuf[pl.ds(j * L, L)] = jnp.zeros((L,), jnp.float32)
        return c

    lax.fori_loop(0, RPW // L, zb, 0)
    pltpu.sync_copy(zbuf, deg_sh.at[pl.ds(sid * RPW, RPW)])
    plsc.subcore_barrier()

    def chunk(k, c):
        pltpu.sync_copy(wv.at[k], deg_sh.at[colv.at[k]], add=True)
        return c

    lax.fori_loop(0, NCH, chunk, 0)
    plsc.subcore_barrier()

    @pl.when(cid == 0)
    def _():
        pltpu.sync_copy(deg_sh.at[pl.ds(sid * RPW, RPW)],
                        dega.at[pl.ds(sid * RPW, RPW)])

    @pl.when(cid == 1)
    def _():
        pltpu.sync_copy(deg_sh.at[pl.ds(sid * RPW, RPW)],
                        degb.at[pl.ds(sid * RPW, RPW)])


def _deg_call(col3, w3):
    f = pl.kernel(
        _deg_body,
        out_type=[jax.ShapeDtypeStruct((NP,), jnp.float32),
                  jax.ShapeDtypeStruct((NP,), jnp.float32)],
        mesh=_mesh(),
        scratch_types=[
            pltpu.VMEM_SHARED((NP,), jnp.float32),
            pltpu.VMEM((NCH, CE), jnp.int32),
            pltpu.VMEM((NCH, CE), jnp.float32),
            pltpu.VMEM((RPW,), jnp.float32),
            pltpu.SemaphoreType.DMA,
        ],
    )
    return f(col3, w3)


NB = 4


def _make_agg_body(H):
    RJ = H // L

    def body(g_hbm, ep, agga, aggb, acc_sh,
             eb0, eb1, eb2, eb3, r0, r1, r2, r3,
             gs0, gs1, gs2, gs3, ss0, ss1, ss2, ss3):
        ebs = (eb0, eb1, eb2, eb3)
        rs = (r0, r1, r2, r3)
        gsem = (gs0, gs1, gs2, gs3)
        ssem = (ss0, ss1, ss2, ss3)
        cid, sid, wid = _worker_ids()

        def zrow(r, c):
            for j in range(RJ):
                r0[r, pl.ds(j * L, L)] = jnp.zeros((L,), jnp.float32)
            return c

        lax.fori_loop(0, CE, zrow, 0)

        def zcp(t, c):
            pltpu.sync_copy(r0, acc_sh.at[pl.ds(sid * RPW + t * CE, CE)])
            return c

        lax.fori_loop(0, RPW // CE, zcp, 0)
        plsc.subcore_barrier()

        def stage(k, b):
            pltpu.sync_copy(ep.at[wid, k], ebs[b])
            pltpu.async_copy(g_hbm.at[ebs[b].at[0]], rs[b], gsem[b])

        def wait_gather(b):
            pltpu.make_async_copy(g_hbm.at[ebs[b].at[0]], rs[b],
                                  gsem[b]).wait()

        def scale(b):
            eb, rb = ebs[b], rs[b]

            def grp(g, c):
                wvec = plsc.bitcast(eb[2, pl.ds(g * L, L)], jnp.float32)
                for t in range(L):
                    ws = wvec[t]
                    e = g * L + t
                    for j in range(RJ):
                        sl = pl.ds(j * L, L)
                        rb[e, sl] = rb[e, sl] * ws
                return c

            lax.fori_loop(0, CE // L, grp, 0)

        def start_scatter(b):
            pltpu.async_copy(rs[b], acc_sh.at[ebs[b].at[1]], ssem[b],
                             add=True)

        def wait_scatter(b):
            pltpu.make_async_copy(rs[b], acc_sh.at[ebs[b].at[1]],
                                  ssem[b]).wait()

        def process(b):
            wait_gather(b)
            scale(b)
            start_scatter(b)

        stage(0, 0)
        stage(1, 1)
        stage(2, 2)
        process(0)
        stage(3, 3)
        process(1)
        wait_scatter(0)
        stage(4, 0)
        process(2)
        wait_scatter(1)
        stage(5, 1)
        process(3)

        def round_(r, c):
            for b in range(NB):
                b2 = (b + 2) % NB
                wait_scatter(b2)
                stage(4 * r + b + 2, b2)
                process(b)
            return c

        lax.fori_loop(1, NCH // NB - 1, round_, 0)

        for b in range(NB):
            b2 = (b + 2) % NB
            wait_scatter(b2)
            if 120 + b + 2 < NCH:
                stage(120 + b + 2, b2)
            process(b)
        wait_scatter(2)
        process(0)
        wait_scatter(3)
        wait_scatter(0)

        plsc.subcore_barrier()

        @pl.when(cid == 0)
        def _():
            pltpu.sync_copy(acc_sh.at[pl.ds(sid * RPW, RPW)],
                            agga.at[pl.ds(sid * RPW, RPW)])

        @pl.when(cid == 1)
        def _():
            pltpu.sync_copy(acc_sh.at[pl.ds(sid * RPW, RPW)],
                            aggb.at[pl.ds(sid * RPW, RPW)])

    return body


def _agg_call(g, ep, H):
    f = pl.kernel(
        _make_agg_body(H),
        out_type=[jax.ShapeDtypeStruct((NP, H), jnp.float32),
                  jax.ShapeDtypeStruct((NP, H), jnp.float32)],
        mesh=_mesh(),
        scratch_types=(
            [pltpu.VMEM_SHARED((NP, H), jnp.float32)]
            + [pltpu.VMEM((3, CE), jnp.int32) for _ in range(NB)]
            + [pltpu.VMEM((CE, H), jnp.float32) for _ in range(NB)]
            + [pltpu.SemaphoreType.DMA] * (2 * NB)
        ),
        compiler_params=pltpu.CompilerParams(needs_layout_passes=False),
    )
    return f(g, ep)


def _fin_body(a2, b2c, g2, dinv_hbm, b2_hbm, idxp, out,
              idxv, ra, rb, rg, dinvv, bv, ob, sem):
    cid, sid, wid = _worker_ids()
    base = wid * QW
    pltpu.sync_copy(idxp.at[pl.ds(base, QW)], idxv)
    pltpu.sync_copy(b2_hbm, bv)
    pltpu.sync_copy(dinv_hbm, dinvv)
    c1 = pltpu.async_copy(a2.at[idxv], ra, sem)
    c2 = pltpu.async_copy(b2c.at[idxv], rb, sem)
    c3 = pltpu.async_copy(g2.at[idxv], rg, sem)
    c1.wait()
    c2.wait()
    c3.wait()

    def rowg(g, c):
        ivec = idxv[pl.ds(g * L, L)]
        dvals = plsc.load_gather(dinvv, [ivec])
        for t in range(L):
            dv = dvals[t]
            r = g * L + t
            for j in range(D // L):
                sl = pl.ds(j * L, L)
                v = dv * (ra[r, sl] + rb[r, sl] + rg[r, sl]) + bv[sl]
                ob[r, sl] = jnp.maximum(v, 0.0)
        return c

    lax.fori_loop(0, QW // L, rowg, 0)
    pltpu.sync_copy(ob, out.at[pl.ds(base, QW)])


def _fin_call(a2, b2c, g2, dinv, b2, idxp):
    f = pl.kernel(
        _fin_body,
        out_type=jax.ShapeDtypeStruct((BQ, D), jnp.float32),
        mesh=_mesh(),
        scratch_types=[
            pltpu.VMEM((QW,), jnp.int32),
            pltpu.VMEM((QW, D), jnp.float32),
            pltpu.VMEM((QW, D), jnp.float32),
            pltpu.VMEM((QW, D), jnp.float32),
            pltpu.VMEM((NP,), jnp.float32),
            pltpu.VMEM((D,), jnp.float32),
            pltpu.VMEM((QW, D), jnp.float32),
            pltpu.SemaphoreType.DMA,
        ],
        compiler_params=pltpu.CompilerParams(needs_layout_passes=False),
    )
    return f(a2, b2c, g2, dinv, b2, idxp)


def _mm1_body(x_ref, w1_ref, dga_ref, dgb_ref, dinv_ref, g1_ref):
    deg = dga_ref[...] + dgb_ref[...] + 1.0
    dinv = jnp.where(deg > 0, lax.rsqrt(jnp.maximum(deg, 1e-12)), 0.0)
    dinv_ref[...] = dinv
    g1_ref[...] = dinv[:, None] * jnp.dot(
        x_ref[...], w1_ref[...], preferred_element_type=jnp.float32)


def _mm1_call(xp, W1, dega, degb):
    return pl.pallas_call(
        _mm1_body,
        grid=(NP // BN,),
        in_specs=[
            pl.BlockSpec((BN, D), lambda i: (i, 0)),
            pl.BlockSpec((D, D), lambda i: (0, 0)),
            pl.BlockSpec((BN,), lambda i: (i,)),
            pl.BlockSpec((BN,), lambda i: (i,)),
        ],
        out_specs=[
            pl.BlockSpec((BN,), lambda i: (i,)),
            pl.BlockSpec((BN, D), lambda i: (i, 0)),
        ],
        out_shape=[
            jax.ShapeDtypeStruct((NP,), jnp.float32),
            jax.ShapeDtypeStruct((NP, D), jnp.float32),
        ],
    )(xp, W1, dega, degb)


def _mm2_body(a_ref, b_ref, g1_ref, dinv_ref, b1_ref, w2_ref, g2_ref):
    dinv = dinv_ref[...][:, None]
    h1 = jnp.maximum(
        dinv * (a_ref[...] + b_ref[...] + g1_ref[...]) + b1_ref[...], 0.0)
    g2_ref[...] = dinv * jnp.dot(
        h1, w2_ref[...], preferred_element_type=jnp.float32)


def _mm2_call(a, b, g1, dinv, b1r, W2):
    return pl.pallas_call(
        _mm2_body,
        grid=(NP // BN,),
        in_specs=[
            pl.BlockSpec((BN, D), lambda i: (i, 0)),
            pl.BlockSpec((BN, D), lambda i: (i, 0)),
            pl.BlockSpec((BN, D), lambda i: (i, 0)),
            pl.BlockSpec((BN,), lambda i: (i,)),
            pl.BlockSpec((1, D), lambda i: (0, 0)),
            pl.BlockSpec((D, D), lambda i: (0, 0)),
        ],
        out_specs=pl.BlockSpec((BN, D), lambda i: (i, 0)),
        out_shape=jax.ShapeDtypeStruct((NP, D), jnp.float32),
    )(a, b, g1, dinv, b1r, W2)


def kernel(x, edge_index, edge_attr, index, W1, b1, W2, b2):
    col3 = edge_index[1].reshape(NW, NCH, CE)
    w3 = edge_attr.reshape(NW, NCH, CE)
    w_bits = lax.bitcast_convert_type(edge_attr, jnp.int32)
    ep = jnp.stack([edge_index[0], edge_index[1], w_bits], axis=0)
    ep = ep.reshape(3, NW, NCH, CE).transpose(1, 2, 0, 3)
    xp = jnp.pad(x, ((0, NP - N), (0, 0)))
    idxp = jnp.pad(index, (0, BQ - NQ))
    b1r = b1.reshape(1, D)
    W2p = jnp.pad(W2, ((0, 0), (0, D - H2)))
    b2p = jnp.pad(b2, (0, D - H2))

    dega, degb = _deg_call(col3, w3)
    dinv, g1 = _mm1_call(xp, W1, dega, degb)
    agg1a, agg1b = _agg_call(g1, ep, D)
    g2 = _mm2_call(agg1a, agg1b, g1, dinv, b1r, W2p)
    agg2a, agg2b = _agg_call(g2, ep, D)
    res = _fin_call(agg2a, agg2b, g2, dinv, b2p, idxp)
    return res[:NQ, :H2]


# speedup vs baseline: 30.1733x; 1.1587x over previous
import functools

import jax
import jax.numpy as jnp
from jax import lax
from jax.experimental import pallas as pl
from jax.experimental.pallas import tpu as pltpu
from jax.experimental.pallas import tpu_sc as plsc

N = 10000
E = 320000
D = 128
H2 = 64
NQ = 1000

NP = 10240
BQ = 1024

L = 16
NC = 2
NS = 16
NW = NC * NS
EW = E // NW
CE = 80
NCH = EW // CE
RPW = NP // NS
QW = BQ // NW

BN = 2048


def _mesh():
    return plsc.VectorSubcoreMesh(core_axis_name="c", subcore_axis_name="s")


def _worker_ids():
    cid = lax.axis_index("c")
    sid = lax.axis_index("s")
    return cid, sid, cid * NS + sid


def _deg_body(col3, w3, dega, degb, deg_sh, colv, wv, zbuf, sem):
    cid, sid, wid = _worker_ids()
    pltpu.sync_copy(col3.at[wid], colv)
    pltpu.sync_copy(w3.at[wid], wv)

    def zb(j, c):
        zbuf[pl.ds(j * L, L)] = jnp.zeros((L,), jnp.float32)
        return c

    lax.fori_loop(0, RPW // L, zb, 0)
    pltpu.sync_copy(zbuf, deg_sh.at[pl.ds(sid * RPW, RPW)])
    plsc.subcore_barrier()

    def chunk(k, c):
        pltpu.sync_copy(wv.at[k], deg_sh.at[colv.at[k]], add=True)
        return c

    lax.fori_loop(0, NCH, chunk, 0)
    plsc.subcore_barrier()

    @pl.when(cid == 0)
    def _():
        pltpu.sync_copy(deg_sh.at[pl.ds(sid * RPW, RPW)],
                        dega.at[pl.ds(sid * RPW, RPW)])

    @pl.when(cid == 1)
    def _():
        pltpu.sync_copy(deg_sh.at[pl.ds(sid * RPW, RPW)],
                        degb.at[pl.ds(sid * RPW, RPW)])


def _deg_call(col3, w3):
    f = pl.kernel(
        _deg_body,
        out_type=[jax.ShapeDtypeStruct((NP,), jnp.float32),
                  jax.ShapeDtypeStruct((NP,), jnp.float32)],
        mesh=_mesh(),
        scratch_types=[
            pltpu.VMEM_SHARED((NP,), jnp.float32),
            pltpu.VMEM((NCH, CE), jnp.int32),
            pltpu.VMEM((NCH, CE), jnp.float32),
            pltpu.VMEM((RPW,), jnp.float32),
            pltpu.SemaphoreType.DMA,
        ],
    )
    return f(col3, w3)


NB = 4


def _make_agg_body(H, HS):
    RJ = H // L
    RJS = HS // L

    def body(g_hbm, ep, agga, aggb, acc_sh,
             eb0, eb1, eb2, eb3, db0, db1, db2, db3, r0, r1, r2, r3,
             es0, es1, es2, es3, gs0, gs1, gs2, gs3, ss0, ss1, ss2, ss3):
        ebs = (eb0, eb1, eb2, eb3)
        dbs = (db0, db1, db2, db3)
        rs = (r0, r1, r2, r3)
        esem = (es0, es1, es2, es3)
        gsem = (gs0, gs1, gs2, gs3)
        ssem = (ss0, ss1, ss2, ss3)
        cid, sid, wid = _worker_ids()

        def zrow(r, c):
            for j in range(RJ):
                r0[r, pl.ds(j * L, L)] = jnp.zeros((L,), jnp.float32)
            return c

        lax.fori_loop(0, CE, zrow, 0)

        def zcp(t, c):
            pltpu.sync_copy(r0, acc_sh.at[pl.ds(sid * RPW + t * CE, CE)])
            return c

        lax.fori_loop(0, RPW // CE, zcp, 0)
        plsc.subcore_barrier()

        def stage_eb(k, b):
            pltpu.async_copy(ep.at[wid, k], ebs[b], esem[b])

        def stage_g(k, b):
            pltpu.make_async_copy(ep.at[wid, k], ebs[b], esem[b]).wait()
            pltpu.async_copy(g_hbm.at[ebs[b].at[0]], rs[b], gsem[b])

        def wait_gather(b):
            pltpu.make_async_copy(g_hbm.at[ebs[b].at[0]], rs[b],
                                  gsem[b]).wait()

        def scale(b):
            eb, db, rb = ebs[b], dbs[b], rs[b]

            @plsc.parallel_loop(0, CE // L, 1, unroll=2)
            def grp(g):
                db[pl.ds(g * L, L)] = eb[1, pl.ds(g * L, L)]
                wvec = plsc.bitcast(eb[2, pl.ds(g * L, L)], jnp.float32)
                for t in range(L):
                    ws = wvec[t]
                    e = g * L + t
                    for j in range(RJS):
                        sl = pl.ds(j * L, L)
                        rb[e, sl] = rb[e, sl] * ws

        def start_scatter(b):
            pltpu.async_copy(rs[b], acc_sh.at[dbs[b]], ssem[b], add=True)

        def wait_scatter(b):
            pltpu.make_async_copy(rs[b], acc_sh.at[dbs[b]], ssem[b]).wait()

        def process(b):
            wait_gather(b)
            scale(b)
            start_scatter(b)

        def step(j, b):
            stage_eb(j + 3, (b + 3) % NB)
            stage_g(j + 2, (b + 2) % NB)
            process(b)

        stage_eb(0, 0)
        stage_eb(1, 1)
        stage_eb(2, 2)
        stage_g(0, 0)
        stage_g(1, 1)
        step(0, 0)
        step(1, 1)
        wait_scatter(0)
        step(2, 2)
        wait_scatter(1)
        step(3, 3)
        wait_scatter(2)
        step(4, 0)
        wait_scatter(3)
        step(5, 1)

        def round_(r, c):
            j0 = 6 + 4 * r
            for t in range(NB):
                b = (2 + t) % NB
                wait_scatter(t)
                step(j0 + t, b)
            return c

        lax.fori_loop(0, (NCH - 9) // NB, round_, 0)

        wait_scatter(0)
        stage_g(124, 0)
        process(2)
        wait_scatter(1)
        process(3)
        wait_scatter(2)
        process(0)
        wait_scatter(3)
        wait_scatter(0)

        plsc.subcore_barrier()

        @pl.when(cid == 0)
        def _():
            pltpu.sync_copy(acc_sh.at[pl.ds(sid * RPW, RPW)],
                            agga.at[pl.ds(sid * RPW, RPW)])

        @pl.when(cid == 1)
        def _():
            pltpu.sync_copy(acc_sh.at[pl.ds(sid * RPW, RPW)],
                            aggb.at[pl.ds(sid * RPW, RPW)])

    return body


def _agg_call(g, ep, H, HS):
    f = pl.kernel(
        _make_agg_body(H, HS),
        out_type=[jax.ShapeDtypeStruct((NP, H), jnp.float32),
                  jax.ShapeDtypeStruct((NP, H), jnp.float32)],
        mesh=_mesh(),
        scratch_types=(
            [pltpu.VMEM_SHARED((NP, H), jnp.float32)]
            + [pltpu.VMEM((3, CE), jnp.int32) for _ in range(NB)]
            + [pltpu.VMEM((CE,), jnp.int32) for _ in range(NB)]
            + [pltpu.VMEM((CE, H), jnp.float32) for _ in range(NB)]
            + [pltpu.SemaphoreType.DMA] * (3 * NB)
        ),
        compiler_params=pltpu.CompilerParams(needs_layout_passes=False),
    )
    return f(g, ep)


def _fin_body(a2, b2c, g2, dinv_hbm, b2_hbm, idxp, out,
              idxv, ra, rb, rg, dinvv, bv, ob, sem):
    cid, sid, wid = _worker_ids()
    base = wid * QW
    pltpu.sync_copy(idxp.at[pl.ds(base, QW)], idxv)
    pltpu.sync_copy(b2_hbm, bv)
    pltpu.sync_copy(dinv_hbm, dinvv)
    c1 = pltpu.async_copy(a2.at[idxv], ra, sem)
    c2 = pltpu.async_copy(b2c.at[idxv], rb, sem)
    c3 = pltpu.async_copy(g2.at[idxv], rg, sem)
    c1.wait()
    c2.wait()
    c3.wait()

    def rowg(g, c):
        ivec = idxv[pl.ds(g * L, L)]
        dvals = plsc.load_gather(dinvv, [ivec])
        for t in range(L):
            dv = dvals[t]
            r = g * L + t
            for j in range(D // L):
                sl = pl.ds(j * L, L)
                v = dv * (ra[r, sl] + rb[r, sl] + rg[r, sl]) + bv[sl]
                ob[r, sl] = jnp.maximum(v, 0.0)
        return c

    lax.fori_loop(0, QW // L, rowg, 0)
    pltpu.sync_copy(ob, out.at[pl.ds(base, QW)])


def _fin_call(a2, b2c, g2, dinv, b2, idxp):
    f = pl.kernel(
        _fin_body,
        out_type=jax.ShapeDtypeStruct((BQ, D), jnp.float32),
        mesh=_mesh(),
        scratch_types=[
            pltpu.VMEM((QW,), jnp.int32),
            pltpu.VMEM((QW, D), jnp.float32),
            pltpu.VMEM((QW, D), jnp.float32),
            pltpu.VMEM((QW, D), jnp.float32),
            pltpu.VMEM((NP,), jnp.float32),
            pltpu.VMEM((D,), jnp.float32),
            pltpu.VMEM((QW, D), jnp.float32),
            pltpu.SemaphoreType.DMA,
        ],
        compiler_params=pltpu.CompilerParams(needs_layout_passes=False),
    )
    return f(a2, b2c, g2, dinv, b2, idxp)


def _mm1_body(x_ref, w1_ref, dga_ref, dgb_ref, dinv_ref, g1_ref):
    deg = dga_ref[...] + dgb_ref[...] + 1.0
    dinv = jnp.where(deg > 0, lax.rsqrt(jnp.maximum(deg, 1e-12)), 0.0)
    dinv_ref[...] = dinv
    g1_ref[...] = dinv[:, None] * jnp.dot(
        x_ref[...], w1_ref[...], preferred_element_type=jnp.float32)


def _mm1_call(xp, W1, dega, degb):
    return pl.pallas_call(
        _mm1_body,
        grid=(NP // BN,),
        in_specs=[
            pl.BlockSpec((BN, D), lambda i: (i, 0)),
            pl.BlockSpec((D, D), lambda i: (0, 0)),
            pl.BlockSpec((BN,), lambda i: (i,)),
            pl.BlockSpec((BN,), lambda i: (i,)),
        ],
        out_specs=[
            pl.BlockSpec((BN,), lambda i: (i,)),
            pl.BlockSpec((BN, D), lambda i: (i, 0)),
        ],
        out_shape=[
            jax.ShapeDtypeStruct((NP,), jnp.float32),
            jax.ShapeDtypeStruct((NP, D), jnp.float32),
        ],
    )(xp, W1, dega, degb)


def _mm2_body(a_ref, b_ref, g1_ref, dinv_ref, b1_ref, w2_ref, g2_ref):
    dinv = dinv_ref[...][:, None]
    h1 = jnp.maximum(
        dinv * (a_ref[...] + b_ref[...] + g1_ref[...]) + b1_ref[...], 0.0)
    g2_ref[...] = dinv * jnp.dot(
        h1, w2_ref[...], preferred_element_type=jnp.float32)


def _mm2_call(a, b, g1, dinv, b1r, W2):
    return pl.pallas_call(
        _mm2_body,
        grid=(NP // BN,),
        in_specs=[
            pl.BlockSpec((BN, D), lambda i: (i, 0)),
            pl.BlockSpec((BN, D), lambda i: (i, 0)),
            pl.BlockSpec((BN, D), lambda i: (i, 0)),
            pl.BlockSpec((BN,), lambda i: (i,)),
            pl.BlockSpec((1, D), lambda i: (0, 0)),
            pl.BlockSpec((D, D), lambda i: (0, 0)),
        ],
        out_specs=pl.BlockSpec((BN, D), lambda i: (i, 0)),
        out_shape=jax.ShapeDtypeStruct((NP, D), jnp.float32),
    )(a, b, g1, dinv, b1r, W2)


def kernel(x, edge_index, edge_attr, index, W1, b1, W2, b2):
    col3 = edge_index[1].reshape(NW, NCH, CE)
    w3 = edge_attr.reshape(NW, NCH, CE)
    w_bits = lax.bitcast_convert_type(edge_attr, jnp.int32)
    ep = jnp.stack([edge_index[0], edge_index[1], w_bits], axis=0)
    ep = ep.reshape(3, NW, NCH, CE).transpose(1, 2, 0, 3)
    xp = jnp.pad(x, ((0, NP - N), (0, 0)))
    idxp = jnp.pad(index, (0, BQ - NQ))
    b1r = b1.reshape(1, D)
    W2p = jnp.pad(W2, ((0, 0), (0, D - H2)))
    b2p = jnp.pad(b2, (0, D - H2))

    dega, degb = _deg_call(col3, w3)
    dinv, g1 = _mm1_call(xp, W1, dega, degb)
    agg1a, agg1b = _agg_call(g1, ep, D, D)
    g2 = _mm2_call(agg1a, agg1b, g1, dinv, b1r, W2p)
    agg2a, agg2b = _agg_call(g2, ep, D, H2)
    res = _fin_call(agg2a, agg2b, g2, dinv, b2p, idxp)
    return res[:NQ, :H2]
